# Initial kernel scaffold; baseline (speedup 1.0000x reference)
import jax
import jax.numpy as jnp
from jax.experimental import pallas as pl


def kernel(h, f_gray, codebook, W_pq, b_pq):
    raise NotImplementedError("write your pallas kernel here")


# speedup vs baseline: 1.0351x; 1.0351x over previous
import functools

import jax
import jax.numpy as jnp
from jax import lax
from jax.experimental import pallas as pl
from jax.experimental.pallas import tpu as pltpu
from jax.experimental.pallas import tpu_sc as plsc

B, C_E, HW = 8, 256, 1024
C_G = 256
N_EMBED = 1024
C_OUT = 512
BETA = 0.25

NC, NS = 2, 16
NW = NC * NS
PER_W = B * HW // NW
CHUNK = 128
NCHUNK = PER_W // CHUNK


def _argmin_kernel(z_ref, cb_ref, idx_ref, loss_ref):
    b = pl.program_id(0)
    z = z_ref[0]
    cb = cb_ref[...]
    scores = lax.dot_general(cb, z, (((1,), (0,)), ((), ())),
                             preferred_element_type=jnp.float32)
    z_norm = jnp.sum(z * z, axis=0, keepdims=True)
    cb_norm = jnp.sum(cb * cb, axis=1, keepdims=True)
    d = (z_norm + cb_norm) - 2.0 * scores
    idx_ref[0] = jnp.argmin(d, axis=0).astype(jnp.int32)[None, :]
    part = jnp.sum(jnp.min(d, axis=0))

    @pl.when(b == 0)
    def _():
        loss_ref[...] = jnp.zeros_like(loss_ref)

    loss_ref[...] += jnp.full((1, 1), 0.0) + part


def _gather_body(cb_hbm, idx_hbm, out_hbm, idx_v, rows_v, sem):
    wid = lax.axis_index("s") * NC + lax.axis_index("c")
    base = wid * PER_W
    pltpu.sync_copy(idx_hbm.at[wid], idx_v)
    for j in range(NCHUNK):
        pltpu.async_copy(cb_hbm.at[idx_v.at[j]], rows_v, sem).wait()
        pltpu.sync_copy(rows_v, out_hbm.at[pl.ds(base + j * CHUNK, CHUNK)])


def _feat_kernel(q_ref, fg_ref, w_ref, b_ref, out_ref):
    q = q_ref[0]
    fg = fg_ref[0]
    w_q = w_ref[:, :C_E]
    w_g = w_ref[:, C_E:]
    t1 = lax.dot_general(w_q, q, (((1,), (1,)), ((), ())),
                         preferred_element_type=jnp.float32)
    t2 = lax.dot_general(w_g, fg, (((1,), (0,)), ((), ())),
                         preferred_element_type=jnp.float32)
    out_ref[0] = t1 + t2 + b_ref[...]


def kernel(h, f_gray, codebook, W_pq, b_pq):
    z = h.reshape(B, C_E, HW)
    fg = f_gray.reshape(B, C_G, HW)

    idx3, loss_sum = pl.pallas_call(
        _argmin_kernel,
        grid=(B,),
        in_specs=[
            pl.BlockSpec((1, C_E, HW), lambda b: (b, 0, 0)),
            pl.BlockSpec((N_EMBED, C_E), lambda b: (0, 0)),
        ],
        out_specs=[
            pl.BlockSpec((1, 1, HW), lambda b: (b, 0, 0)),
            pl.BlockSpec((1, 1), lambda b: (0, 0)),
        ],
        out_shape=[
            jax.ShapeDtypeStruct((B, 1, HW), jnp.int32),
            jax.ShapeDtypeStruct((1, 1), jnp.float32),
        ],
    )(z, codebook)

    color_idx = idx3.reshape(B, HW)
    emb_loss = ((1.0 + BETA) / (B * HW * C_E)) * loss_sum[0, 0]

    idx_w = idx3.reshape(NW, NCHUNK, CHUNK)

    gather = pl.kernel(
        _gather_body,
        out_type=jax.ShapeDtypeStruct((B * HW, C_E), jnp.float32),
        mesh=plsc.VectorSubcoreMesh(core_axis_name="c", subcore_axis_name="s"),
        scratch_types=[
            pltpu.VMEM((NCHUNK, CHUNK), jnp.int32),
            pltpu.VMEM((CHUNK, C_E), jnp.float32),
            pltpu.SemaphoreType.DMA,
        ],
    )
    quant_rows = gather(codebook, idx_w)

    feat = pl.pallas_call(
        _feat_kernel,
        grid=(B,),
        in_specs=[
            pl.BlockSpec((1, HW, C_E), lambda b: (b, 0, 0)),
            pl.BlockSpec((1, C_G, HW), lambda b: (b, 0, 0)),
            pl.BlockSpec((C_OUT, C_E + C_G), lambda b: (0, 0)),
            pl.BlockSpec((C_OUT, 1), lambda b: (0, 0)),
        ],
        out_specs=pl.BlockSpec((1, C_OUT, HW), lambda b: (b, 0, 0)),
        out_shape=jax.ShapeDtypeStruct((B, C_OUT, HW), jnp.float32),
    )(quant_rows.reshape(B, HW, C_E), fg, W_pq, b_pq.reshape(C_OUT, 1))

    feat = feat.reshape(B, C_OUT, 32, 32)
    return feat, emb_loss, color_idx


# speedup vs baseline: 1.4600x; 1.4106x over previous
import jax
import jax.numpy as jnp
from jax import lax
from jax.experimental import pallas as pl
from jax.experimental.pallas import tpu as pltpu
from jax.experimental.pallas import tpu_sc as plsc

B, C_E, HW = 8, 256, 1024
C_G = 256
N_EMBED = 1024
C_OUT = 512
BETA = 0.25
S = B * HW
BLK = 1024
NBLK = S // BLK

NC, NS = 2, 16
NW = NC * NS
PER_W = S // NW
CHUNK = 128
NCHUNK = PER_W // CHUNK


def _argmin_kernel(z_ref, cb_ref, idx_ref, loss_ref):
    i = pl.program_id(0)
    z = z_ref[...]
    cb = cb_ref[...]
    scores = lax.dot_general(cb, z, (((1,), (1,)), ((), ())),
                             preferred_element_type=jnp.float32)
    zz = z * z
    ones = jnp.ones((1, C_E), dtype=jnp.float32)
    z_norm = lax.dot_general(ones, zz, (((1,), (1,)), ((), ())),
                             preferred_element_type=jnp.float32)
    cb_norm = jnp.sum(cb * cb, axis=1, keepdims=True)
    d = (z_norm + cb_norm) - 2.0 * scores
    idx_ref[0] = jnp.argmin(d, axis=0).astype(jnp.int32)[None, :]
    part = jnp.sum(jnp.min(d, axis=0))

    @pl.when(i == 0)
    def _():
        loss_ref[...] = jnp.zeros_like(loss_ref)

    loss_ref[...] += jnp.full((1, 1), 0.0) + part


def _gather_body(cb_hbm, idx_hbm, out_hbm, idx_v, rows_v, sem):
    wid = lax.axis_index("s") * NC + lax.axis_index("c")
    base = wid * PER_W
    pltpu.sync_copy(idx_hbm.at[wid], idx_v)
    for j in range(NCHUNK):
        pltpu.async_copy(cb_hbm.at[idx_v.at[j]], rows_v, sem).wait()
        pltpu.sync_copy(rows_v, out_hbm.at[pl.ds(base + j * CHUNK, CHUNK)])


def _feat_kernel(q_ref, fg_ref, w_ref, b_ref, out_ref):
    q = q_ref[...]
    fg = fg_ref[...]
    w_q = w_ref[:, :C_E]
    w_g = w_ref[:, C_E:]
    t1 = lax.dot_general(q, w_q, (((1,), (1,)), ((), ())),
                         preferred_element_type=jnp.float32)
    t2 = lax.dot_general(fg, w_g, (((1,), (1,)), ((), ())),
                         preferred_element_type=jnp.float32)
    out_ref[...] = t1 + t2 + b_ref[...]


def kernel(h, f_gray, codebook, W_pq, b_pq):
    z_flat = jnp.transpose(h.reshape(B, C_E, HW), (0, 2, 1)).reshape(S, C_E)
    fg_flat = jnp.transpose(f_gray.reshape(B, C_G, HW), (0, 2, 1)).reshape(S, C_G)

    idx3, loss_sum = pl.pallas_call(
        _argmin_kernel,
        grid=(NBLK,),
        in_specs=[
            pl.BlockSpec((BLK, C_E), lambda i: (i, 0)),
            pl.BlockSpec((N_EMBED, C_E), lambda i: (0, 0)),
        ],
        out_specs=[
            pl.BlockSpec((1, 1, BLK), lambda i: (i, 0, 0)),
            pl.BlockSpec((1, 1), lambda i: (0, 0)),
        ],
        out_shape=[
            jax.ShapeDtypeStruct((NBLK, 1, BLK), jnp.int32),
            jax.ShapeDtypeStruct((1, 1), jnp.float32),
        ],
    )(z_flat, codebook)

    color_idx = idx3.reshape(B, HW)
    emb_loss = ((1.0 + BETA) / (S * C_E)) * loss_sum[0, 0]

    idx_w = idx3.reshape(NW, NCHUNK, CHUNK)

    gather = pl.kernel(
        _gather_body,
        out_type=jax.ShapeDtypeStruct((S, C_E), jnp.float32),
        mesh=plsc.VectorSubcoreMesh(core_axis_name="c", subcore_axis_name="s"),
        scratch_types=[
            pltpu.VMEM((NCHUNK, CHUNK), jnp.int32),
            pltpu.VMEM((CHUNK, C_E), jnp.float32),
            pltpu.SemaphoreType.DMA,
        ],
    )
    quant_rows = gather(codebook, idx_w)

    feat_flat = pl.pallas_call(
        _feat_kernel,
        grid=(NBLK,),
        in_specs=[
            pl.BlockSpec((BLK, C_E), lambda i: (i, 0)),
            pl.BlockSpec((BLK, C_G), lambda i: (i, 0)),
            pl.BlockSpec((C_OUT, C_E + C_G), lambda i: (0, 0)),
            pl.BlockSpec((1, C_OUT), lambda i: (0, 0)),
        ],
        out_specs=pl.BlockSpec((BLK, C_OUT), lambda i: (i, 0)),
        out_shape=jax.ShapeDtypeStruct((S, C_OUT), jnp.float32),
    )(quant_rows, fg_flat, W_pq, b_pq.reshape(1, C_OUT))

    feat = jnp.transpose(feat_flat.reshape(B, HW, C_OUT), (0, 2, 1))
    feat = feat.reshape(B, C_OUT, 32, 32)
    return feat, emb_loss, color_idx
